# Initial kernel scaffold; baseline (speedup 1.0000x reference)
import jax
import jax.numpy as jnp
from jax.experimental import pallas as pl


def kernel(x, y, calib):
    raise NotImplementedError("write your pallas kernel here")


# speedup vs baseline: 23.7122x; 23.7122x over previous
import functools

import jax
import jax.numpy as jnp
from jax import lax
from jax.experimental import pallas as pl
from jax.experimental.pallas import tpu as pltpu
from jax.experimental.pallas import tpu_sc as plsc

_N = 1_000_000
_C = 10
_NBINS = 50
_GP = 64
_ROWLEN = _GP * _C
_NROWS = _N // _GP
_RB = 512
_GRID_A = -(-_NROWS // _RB)

_NSLOT = 1281
_NSLOT_PAD = 1288
_HWORDS = _NSLOT_PAD * 16

_NTILES = 32
_CHUNK = 8000
_NCHUNK = (_N * _C) // _CHUNK


def _stage_a_body(x_ref, y_ref, g_ref, gt_ref, cls_ref, addr_ref, bl_ref):
    i = pl.program_id(0)
    row0 = i * _RB
    riota = lax.broadcasted_iota(jnp.int32, (_RB, 1), 0) + row0
    valid = riota < _NROWS
    xb = jnp.where(valid, x_ref[...], 0.0)
    yb = jnp.where(valid, y_ref[...], 0.0)

    e = jnp.exp(xb)
    s = jnp.dot(e, g_ref[...], precision=lax.Precision.HIGHEST)
    r = 1.0 / s
    rx = jnp.dot(r, gt_ref[...], precision=lax.Precision.HIGHEST)
    ye = jnp.dot(yb, gt_ref[...], precision=lax.Precision.HIGHEST)
    p = e * rx

    idf = jnp.floor(p * 50.0)
    elo = idf * 0.02
    ehi = (idf + 1.0) * 0.02
    idf = idf + (p >= ehi).astype(jnp.float32) - (p < elo).astype(jnp.float32)

    clsf = cls_ref[...]
    w = (ye == clsf).astype(jnp.float32)
    slotf = clsf * 64.0 + idf + w * 640.0
    slotf = jnp.where(valid, slotf, float(_NSLOT - 1))
    addr_ref[...] = (slotf * 16.0).astype(jnp.int32)

    d = xb - ye
    part = jnp.sum(d * d).reshape(1, 1)

    @pl.when(i == 0)
    def _():
        bl_ref[...] = jnp.zeros_like(bl_ref)

    bl_ref[...] += part


def _run_stage_a(xr, yr, g, gt, cls640, interpret=False):
    return pl.pallas_call(
        _stage_a_body,
        grid=(_GRID_A,),
        in_specs=[
            pl.BlockSpec((_RB, _ROWLEN), lambda i: (i, 0)),
            pl.BlockSpec((_RB, _GP), lambda i: (i, 0)),
            pl.BlockSpec((_ROWLEN, _GP), lambda i: (0, 0)),
            pl.BlockSpec((_GP, _ROWLEN), lambda i: (0, 0)),
            pl.BlockSpec((1, _ROWLEN), lambda i: (0, 0)),
        ],
        out_specs=[
            pl.BlockSpec((_RB, _ROWLEN), lambda i: (i, 0)),
            pl.BlockSpec((1, 1), lambda i: (0, 0)),
        ],
        out_shape=[
            jax.ShapeDtypeStruct((_NROWS, _ROWLEN), jnp.int32),
            jax.ShapeDtypeStruct((1, 1), jnp.float32),
        ],
        interpret=interpret,
    )(xr, yr, g, gt, cls640)


def _sc_hist_body(a_hbm, out_hbm, buf, hist):
    cid = lax.axis_index("c")
    sid = lax.axis_index("s")
    wid = sid * 2 + cid

    zeros = jnp.zeros((16,), jnp.float32)

    def zero_body(k, _):
        hist[pl.ds(k * 16, 16)] = zeros
        return 0

    lax.fori_loop(0, _HWORDS // 16, zero_body, 0)

    lane_iota = lax.iota(jnp.int32, 16)
    ones16 = jnp.ones((16,), jnp.float32)

    nch = jnp.where(wid < _NCHUNK - (_NCHUNK // _NTILES) * _NTILES,
                    _NCHUNK // _NTILES + 1, _NCHUNK // _NTILES)

    def chunk_body(t, _):
        g = wid + _NTILES * t
        pltpu.sync_copy(a_hbm.at[pl.ds(g * _CHUNK, _CHUNK)], buf)

        def grp_body(k, _2):
            for u in range(4):
                base = buf[pl.ds((k * 4 + u) * 16, 16)]
                plsc.addupdate_scatter(hist, [base + lane_iota], ones16)
            return 0

        lax.fori_loop(0, _CHUNK // 64, grp_body, 0)
        return 0

    lax.fori_loop(0, nch, chunk_body, 0)
    pltpu.sync_copy(hist, out_hbm.at[wid])


def _run_stage_b(addr_flat):
    k = functools.partial(
        pl.kernel,
        mesh=plsc.VectorSubcoreMesh(core_axis_name="c", subcore_axis_name="s"),
        out_type=jax.ShapeDtypeStruct((_NTILES, _HWORDS), jnp.float32),
        scratch_types=[
            pltpu.VMEM((_CHUNK,), jnp.int32),
            pltpu.VMEM((_HWORDS,), jnp.float32),
        ],
        compiler_params=pltpu.CompilerParams(needs_layout_passes=False),
    )(_sc_hist_body)
    return k(addr_flat)


def _sigmoid(z):
    return 1.0 / (1.0 + jnp.exp(-z))


def _stage_c_body(h_ref, calib_ref, bl_ref, out_ref, acc_ref):
    w = pl.program_id(0)

    @pl.when(w == 0)
    def _():
        acc_ref[...] = h_ref[...]

    @pl.when(w > 0)
    def _():
        acc_ref[...] += h_ref[...]

    @pl.when(w == _NTILES - 1)
    def _():
        hist = jnp.sum(acc_ref[...], axis=1, keepdims=True)
        ece = jnp.float32(0.0)
        for c in range(1, _C - 1):
            tru = hist[640 + c * 64: 640 + c * 64 + _NBINS, 0:1]
            tot = tru + hist[c * 64: c * 64 + _NBINS, 0:1]
            ratio = _sigmoid(tru) / _sigmoid(tot)
            diff = _sigmoid(calib_ref[:, c:c + 1]) - ratio
            ece = ece + jnp.sum(diff * diff) * (1.0 / _NBINS)
        out_ref[...] = bl_ref[...] * (1.0 / (_N * _C)) + ece


def _run_stage_c(hparts, calib, bl, interpret=False):
    return pl.pallas_call(
        _stage_c_body,
        grid=(_NTILES,),
        in_specs=[
            pl.BlockSpec((_NSLOT_PAD, 16), lambda w: (w, 0)),
            pl.BlockSpec((_NBINS, _C), lambda w: (0, 0)),
            pl.BlockSpec((1, 1), lambda w: (0, 0)),
        ],
        out_specs=pl.BlockSpec((1, 1), lambda w: (0, 0)),
        out_shape=jax.ShapeDtypeStruct((1, 1), jnp.float32),
        scratch_shapes=[pltpu.VMEM((_NSLOT_PAD, 16), jnp.float32)],
        interpret=interpret,
    )(hparts, calib, bl)


def _constants():
    gi = jnp.arange(_ROWLEN, dtype=jnp.int32) // _C
    g = (gi[:, None] == jnp.arange(_GP, dtype=jnp.int32)[None, :]).astype(jnp.float32)
    gt = g.T
    cls640 = (jnp.arange(_ROWLEN, dtype=jnp.int32) % _C).astype(jnp.float32)[None, :]
    return g, gt, cls640


def kernel(x, y, calib):
    xr = x.reshape(_NROWS, _ROWLEN)
    yr = y.reshape(_NROWS, _GP)
    g, gt, cls640 = _constants()
    addr, bl = _run_stage_a(xr, yr, g, gt, cls640)
    hparts = _run_stage_b(addr.reshape(_N * _C))
    out = _run_stage_c(hparts.reshape(_NTILES * _NSLOT_PAD, 16), calib, bl)
    return out[0, 0]


# speedup vs baseline: 28.7215x; 1.2113x over previous
import functools

import jax
import jax.numpy as jnp
from jax import lax
from jax.experimental import pallas as pl
from jax.experimental.pallas import tpu as pltpu
from jax.experimental.pallas import tpu_sc as plsc

_N = 1_000_000
_C = 10
_NBINS = 50
_GP = 64
_ROWLEN = _GP * _C
_NROWS = _N // _GP
_RB = 512
_GRID_A = -(-_NROWS // _RB)
_NROWS_PAD = _GRID_A * _RB

_NSLOT = 1281
_NSLOT_PAD = 1288
_HWORDS = _NSLOT_PAD * 16

_NTILES = 32
_CROWS = 32
_NCHUNK = _NROWS_PAD // _CROWS
_TPT = -(-_NCHUNK // _NTILES)


def _stage_a_body(x_ref, y_ref, g_ref, gt_ref, cls_ref, addr_ref, bl_ref):
    i = pl.program_id(0)
    row0 = i * _RB
    riota = lax.broadcasted_iota(jnp.int32, (_RB, 1), 0) + row0
    valid = riota < _NROWS
    xb = jnp.where(valid, x_ref[...], 0.0)
    yb = jnp.where(valid, y_ref[...], 0.0)

    e = jnp.exp(xb)

    def _split3(v):
        hi = v.astype(jnp.bfloat16)
        r1 = v - hi.astype(jnp.float32)
        mid = r1.astype(jnp.bfloat16)
        lo = (r1 - mid.astype(jnp.float32)).astype(jnp.bfloat16)
        return hi, mid, lo

    def _dot3(v, m_ref):
        m = m_ref[...]
        hi, mid, lo = _split3(v)
        acc = jnp.dot(hi, m, preferred_element_type=jnp.float32)
        acc += jnp.dot(mid, m, preferred_element_type=jnp.float32)
        acc += jnp.dot(lo, m, preferred_element_type=jnp.float32)
        return acc

    s = _dot3(e, g_ref)
    r = 1.0 / s
    rx = _dot3(r, gt_ref)
    ye = jnp.dot(yb.astype(jnp.bfloat16), gt_ref[...],
                 preferred_element_type=jnp.float32)
    p = e * rx

    idf = jnp.floor(p * 50.0)
    elo = idf * 0.02
    ehi = (idf + 1.0) * 0.02
    idf = idf + (p >= ehi).astype(jnp.float32) - (p < elo).astype(jnp.float32)

    clsf = cls_ref[...]
    w = (ye == clsf).astype(jnp.float32)
    slotf = clsf * 64.0 + idf + w * 640.0
    slotf = jnp.where(valid, slotf, float(_NSLOT - 1))
    addr_ref[...] = (slotf * 16.0).astype(jnp.int32)

    d = xb - ye
    part = jnp.sum(d * d).reshape(1, 1)

    @pl.when(i == 0)
    def _():
        bl_ref[...] = jnp.zeros_like(bl_ref)

    bl_ref[...] += part


def _run_stage_a(xr, yr, g, gt, cls640, interpret=False):
    return pl.pallas_call(
        _stage_a_body,
        grid=(_GRID_A,),
        in_specs=[
            pl.BlockSpec((_RB, _ROWLEN), lambda i: (i, 0)),
            pl.BlockSpec((_RB, _GP), lambda i: (i, 0)),
            pl.BlockSpec((_ROWLEN, _GP), lambda i: (0, 0)),
            pl.BlockSpec((_GP, _ROWLEN), lambda i: (0, 0)),
            pl.BlockSpec((1, _ROWLEN), lambda i: (0, 0)),
        ],
        out_specs=[
            pl.BlockSpec((_RB, _ROWLEN), lambda i: (i, 0)),
            pl.BlockSpec((1, 1), lambda i: (0, 0)),
        ],
        out_shape=[
            jax.ShapeDtypeStruct((_NROWS_PAD, _ROWLEN), jnp.int32),
            jax.ShapeDtypeStruct((1, 1), jnp.float32),
        ],
        interpret=interpret,
    )(xr, yr, g, gt, cls640)


def _sc_hist_body(a_hbm, out_hbm, buf0, buf1, hist, sem0, sem1):
    cid = lax.axis_index("c")
    sid = lax.axis_index("s")
    wid = sid * 2 + cid

    zeros = jnp.zeros((16,), jnp.float32)

    def zero_body(k, _):
        hist[pl.ds(k * 16, 16)] = zeros
        return 0

    lax.fori_loop(0, _HWORDS // 16, zero_body, 0)

    lane_iota = lax.iota(jnp.int32, 16)
    ones16 = jnp.ones((16,), jnp.float32)

    def dma_start(t, buf, sem):
        g = jnp.minimum(wid + _NTILES * t, _NCHUNK - 1)
        return pltpu.async_copy(a_hbm.at[pl.ds(g * _CROWS, _CROWS)], buf, sem)

    def process(buf, t):
        mvec = jnp.full((16,), wid + _NTILES * t < _NCHUNK)

        def row_body(r, _):
            def col_body(c4, _2):
                for u in range(4):
                    s = buf[r, pl.ds((c4 * 4 + u) * 16, 16)]
                    plsc.addupdate_scatter(hist, [s + lane_iota], ones16,
                                           mask=mvec)
                return 0

            lax.fori_loop(0, _ROWLEN // 64, col_body, 0)
            return 0

        lax.fori_loop(0, _CROWS, row_body, 0)

    bufs = (buf0, buf1)
    sems = (sem0, sem1)
    h = dma_start(0, buf0, sem0)
    for t in range(_TPT):
        if t + 1 < _TPT:
            hn = dma_start(t + 1, bufs[(t + 1) % 2], sems[(t + 1) % 2])
        h.wait()
        process(bufs[t % 2], t)
        if t + 1 < _TPT:
            h = hn
    pltpu.sync_copy(hist, out_hbm.at[wid])


def _run_stage_b(addr2d):
    k = functools.partial(
        pl.kernel,
        mesh=plsc.VectorSubcoreMesh(core_axis_name="c", subcore_axis_name="s"),
        out_type=jax.ShapeDtypeStruct((_NTILES, _HWORDS), jnp.float32),
        scratch_types=[
            pltpu.VMEM((_CROWS, _ROWLEN), jnp.int32),
            pltpu.VMEM((_CROWS, _ROWLEN), jnp.int32),
            pltpu.VMEM((_HWORDS,), jnp.float32),
            pltpu.SemaphoreType.DMA,
            pltpu.SemaphoreType.DMA,
        ],
        compiler_params=pltpu.CompilerParams(needs_layout_passes=False),
    )(_sc_hist_body)
    return k(addr2d)


def _sigmoid(z):
    return 1.0 / (1.0 + jnp.exp(-z))


def _stage_c_body(h_ref, calib_ref, bl_ref, out_ref, acc_ref):
    w = pl.program_id(0)

    @pl.when(w == 0)
    def _():
        acc_ref[...] = h_ref[...]

    @pl.when(w > 0)
    def _():
        acc_ref[...] += h_ref[...]

    @pl.when(w == _NTILES - 1)
    def _():
        hist = jnp.sum(acc_ref[...], axis=1, keepdims=True)
        ece = jnp.float32(0.0)
        for c in range(1, _C - 1):
            tru = hist[640 + c * 64: 640 + c * 64 + _NBINS, 0:1]
            tot = tru + hist[c * 64: c * 64 + _NBINS, 0:1]
            ratio = _sigmoid(tru) / _sigmoid(tot)
            diff = _sigmoid(calib_ref[:, c:c + 1]) - ratio
            ece = ece + jnp.sum(diff * diff) * (1.0 / _NBINS)
        out_ref[...] = bl_ref[...] * (1.0 / (_N * _C)) + ece


def _run_stage_c(hparts, calib, bl, interpret=False):
    return pl.pallas_call(
        _stage_c_body,
        grid=(_NTILES,),
        in_specs=[
            pl.BlockSpec((_NSLOT_PAD, 16), lambda w: (w, 0)),
            pl.BlockSpec((_NBINS, _C), lambda w: (0, 0)),
            pl.BlockSpec((1, 1), lambda w: (0, 0)),
        ],
        out_specs=pl.BlockSpec((1, 1), lambda w: (0, 0)),
        out_shape=jax.ShapeDtypeStruct((1, 1), jnp.float32),
        scratch_shapes=[pltpu.VMEM((_NSLOT_PAD, 16), jnp.float32)],
        interpret=interpret,
    )(hparts, calib, bl)


def _constants():
    gi = jnp.arange(_ROWLEN, dtype=jnp.int32) // _C
    g = (gi[:, None] == jnp.arange(_GP, dtype=jnp.int32)[None, :]).astype(jnp.bfloat16)
    gt = g.T
    cls640 = (jnp.arange(_ROWLEN, dtype=jnp.int32) % _C).astype(jnp.float32)[None, :]
    return g, gt, cls640


def kernel(x, y, calib):
    xr = x.reshape(_NROWS, _ROWLEN)
    yr = y.reshape(_NROWS, _GP)
    g, gt, cls640 = _constants()
    addr, bl = _run_stage_a(xr, yr, g, gt, cls640)
    hparts = _run_stage_b(addr)
    out = _run_stage_c(hparts.reshape(_NTILES * _NSLOT_PAD, 16), calib, bl)
    return out[0, 0]


# speedup vs baseline: 75.5522x; 2.6305x over previous
import functools

import jax
import jax.numpy as jnp
from jax import lax
from jax.experimental import pallas as pl
from jax.experimental.pallas import tpu as pltpu
from jax.experimental.pallas import tpu_sc as plsc

_N = 1_000_000
_C = 10
_NBINS = 50

_LB = 32_000
_GRID_A = 32
_NPAD = _GRID_A * _LB

_NSLOT = 1281
_NSLOT_PAD = 1288
_HWORDS = _NSLOT_PAD * 16

_NTILES = 32
_CCOLS = 3_200
_CPT = _NPAD // (_NTILES * _CCOLS)


def _stage_a_body(x_ref, y_ref, addr_ref, bl_ref):
    i = pl.program_id(0)
    col0 = i * _LB
    colv = (lax.broadcasted_iota(jnp.int32, (1, _LB), 1) + col0) < _N
    xb = x_ref[...]
    yb = y_ref[...]

    e = jnp.exp(xb)
    s = jnp.sum(e, axis=0, keepdims=True)
    p = e * (1.0 / s)

    ids = (p * 800.0).astype(jnp.int32) & ~15
    clsi = lax.broadcasted_iota(jnp.int32, (_C, 1), 0)
    clsf = clsi.astype(jnp.float32)
    w = (yb == clsf)
    base = jnp.where(w, clsi * 1024 + 10240, clsi * 1024)
    slot = jnp.where(colv, ids + base, 16 * (_NSLOT - 1))
    addr_ref[...] = slot

    d = jnp.where(colv, xb - yb, 0.0)
    part = jnp.sum(d * d).reshape(1, 1)

    @pl.when(i == 0)
    def _():
        bl_ref[...] = jnp.zeros_like(bl_ref)

    bl_ref[...] += part


def _run_stage_a(xt, yt, interpret=False):
    return pl.pallas_call(
        _stage_a_body,
        grid=(_GRID_A,),
        in_specs=[
            pl.BlockSpec((_C, _LB), lambda i: (0, i)),
            pl.BlockSpec((1, _LB), lambda i: (0, i)),
        ],
        out_specs=[
            pl.BlockSpec((_C, _LB), lambda i: (0, i)),
            pl.BlockSpec((1, 1), lambda i: (0, 0)),
        ],
        out_shape=[
            jax.ShapeDtypeStruct((_C, _NPAD), jnp.int32),
            jax.ShapeDtypeStruct((1, 1), jnp.float32),
        ],
        interpret=interpret,
    )(xt, yt)


def _sc_hist_body(a_hbm, out_hbm, buf0, buf1, hist, sem0, sem1):
    cid = lax.axis_index("c")
    sid = lax.axis_index("s")
    wid = sid * 2 + cid

    zeros = jnp.zeros((16,), jnp.float32)

    def zero_body(k, _):
        hist[pl.ds(k * 16, 16)] = zeros
        return 0

    lax.fori_loop(0, _HWORDS // 16, zero_body, 0)

    lane_iota = lax.iota(jnp.int32, 16)
    ones16 = jnp.ones((16,), jnp.float32)

    def dma_start(k, buf, sem):
        col = (k * _NTILES + wid) * _CCOLS
        return pltpu.async_copy(
            a_hbm.at[pl.ds(0, _C), pl.ds(col, _CCOLS)], buf, sem)

    def process(buf):
        def row_body(r, _):
            def col_body(c4, _2):
                for u in range(4):
                    s = buf[r, pl.ds((c4 * 4 + u) * 16, 16)]
                    plsc.addupdate_scatter(hist, [s + lane_iota], ones16)
                return 0

            lax.fori_loop(0, _CCOLS // 64, col_body, 0)
            return 0

        lax.fori_loop(0, _C, row_body, 0)

    bufs = (buf0, buf1)
    sems = (sem0, sem1)
    h = dma_start(0, buf0, sem0)
    for k in range(_CPT):
        if k + 1 < _CPT:
            hn = dma_start(k + 1, bufs[(k + 1) % 2], sems[(k + 1) % 2])
        h.wait()
        process(bufs[k % 2])
        if k + 1 < _CPT:
            h = hn
    pltpu.sync_copy(hist, out_hbm.at[wid])


def _run_stage_b(addr2d):
    k = functools.partial(
        pl.kernel,
        mesh=plsc.VectorSubcoreMesh(core_axis_name="c", subcore_axis_name="s"),
        out_type=jax.ShapeDtypeStruct((_NTILES, _HWORDS), jnp.float32),
        scratch_types=[
            pltpu.VMEM((_C, _CCOLS), jnp.int32),
            pltpu.VMEM((_C, _CCOLS), jnp.int32),
            pltpu.VMEM((_HWORDS,), jnp.float32),
            pltpu.SemaphoreType.DMA,
            pltpu.SemaphoreType.DMA,
        ],
        compiler_params=pltpu.CompilerParams(needs_layout_passes=False),
    )(_sc_hist_body)
    return k(addr2d)


def _sigmoid(z):
    return 1.0 / (1.0 + jnp.exp(-z))


def _stage_c_body(h_ref, calib_ref, bl_ref, out_ref, acc_ref):
    w = pl.program_id(0)

    @pl.when(w == 0)
    def _():
        acc_ref[...] = h_ref[...]

    @pl.when(w > 0)
    def _():
        acc_ref[...] += h_ref[...]

    @pl.when(w == _NTILES - 1)
    def _():
        hist = jnp.sum(acc_ref[...], axis=1, keepdims=True)
        ece = jnp.float32(0.0)
        for c in range(1, _C - 1):
            tru = hist[640 + c * 64: 640 + c * 64 + _NBINS, 0:1]
            tot = tru + hist[c * 64: c * 64 + _NBINS, 0:1]
            ratio = _sigmoid(tru) / _sigmoid(tot)
            diff = _sigmoid(calib_ref[:, c:c + 1]) - ratio
            ece = ece + jnp.sum(diff * diff) * (1.0 / _NBINS)
        out_ref[...] = bl_ref[...] * (1.0 / (_N * _C)) + ece


def _run_stage_c(hparts, calib, bl, interpret=False):
    return pl.pallas_call(
        _stage_c_body,
        grid=(_NTILES,),
        in_specs=[
            pl.BlockSpec((_NSLOT_PAD, 16), lambda w: (w, 0)),
            pl.BlockSpec((_NBINS, _C), lambda w: (0, 0)),
            pl.BlockSpec((1, 1), lambda w: (0, 0)),
        ],
        out_specs=pl.BlockSpec((1, 1), lambda w: (0, 0)),
        out_shape=jax.ShapeDtypeStruct((1, 1), jnp.float32),
        scratch_shapes=[pltpu.VMEM((_NSLOT_PAD, 16), jnp.float32)],
        interpret=interpret,
    )(hparts, calib, bl)


def kernel(x, y, calib):
    xt = x.T
    yt = y.T
    addr, bl = _run_stage_a(xt, yt)
    hparts = _run_stage_b(addr)
    out = _run_stage_c(hparts.reshape(_NTILES * _NSLOT_PAD, 16), calib, bl)
    return out[0, 0]


# speedup vs baseline: 91.2788x; 1.2082x over previous
import functools

import jax
import jax.numpy as jnp
from jax import lax
from jax.experimental import pallas as pl
from jax.experimental.pallas import tpu as pltpu
from jax.experimental.pallas import tpu_sc as plsc

_N = 1_000_000
_C = 10
_NBINS = 50

_LB = 32_000
_GRID_A = 32
_NPAD = _GRID_A * _LB

_NSLOT = 1281
_NSLOT_PAD = 1296
_HWORDS = _NSLOT_PAD * 16

_NTILES = 32
_CCOLS = 3_200
_CPT = _NPAD // (_NTILES * _CCOLS)


def _stage_a_body(x_ref, y_ref, addr_ref, bl_ref):
    i = pl.program_id(0)
    col0 = i * _LB
    colv = (lax.broadcasted_iota(jnp.int32, (1, _LB), 1) + col0) < _N
    xb = x_ref[...]
    yb = y_ref[...]

    e = jnp.exp(xb)
    s = jnp.sum(e, axis=0, keepdims=True)
    p = e * (1.0 / s)

    ids = (p * 800.0).astype(jnp.int32) & ~15
    clsi = lax.broadcasted_iota(jnp.int32, (_C, 1), 0)
    clsf = clsi.astype(jnp.float32)
    w = (yb == clsf)
    base = jnp.where(w, clsi * 1024 + 10240, clsi * 1024)
    slot = jnp.where(colv, ids + base, 16 * (_NSLOT - 1))
    addr_ref[...] = slot

    d = jnp.where(colv, xb - yb, 0.0)
    part = jnp.sum(d * d).reshape(1, 1)

    @pl.when(i == 0)
    def _():
        bl_ref[...] = jnp.zeros_like(bl_ref)

    bl_ref[...] += part


def _run_stage_a(xt, yt, interpret=False):
    return pl.pallas_call(
        _stage_a_body,
        grid=(_GRID_A,),
        in_specs=[
            pl.BlockSpec((_C, _LB), lambda i: (0, i)),
            pl.BlockSpec((1, _LB), lambda i: (0, i)),
        ],
        out_specs=[
            pl.BlockSpec((_C, _LB), lambda i: (0, i)),
            pl.BlockSpec((1, 1), lambda i: (0, 0)),
        ],
        out_shape=[
            jax.ShapeDtypeStruct((_C, _NPAD), jnp.int32),
            jax.ShapeDtypeStruct((1, 1), jnp.float32),
        ],
        interpret=interpret,
    )(xt, yt)


def _sc_hist_body(a_hbm, out_hbm, buf0, buf1, hist, folded, sem0, sem1):
    cid = lax.axis_index("c")
    sid = lax.axis_index("s")
    wid = sid * 2 + cid

    zeros = jnp.zeros((16,), jnp.float32)

    def zero_body(k, _):
        for u in range(4):
            hist[pl.ds((k * 4 + u) * 16, 16)] = zeros
        return 0

    lax.fori_loop(0, _HWORDS // 64, zero_body, 0)

    lane_iota = lax.iota(jnp.int32, 16)
    ones16 = jnp.ones((16,), jnp.float32)

    def dma_start(k, buf, sem):
        col = (k * _NTILES + wid) * _CCOLS
        return pltpu.async_copy(
            a_hbm.at[pl.ds(0, _C), pl.ds(col, _CCOLS)], buf, sem)

    def process(buf):
        def row_body(r, _):
            def col_body(c8, _2):
                for u in range(8):
                    s = buf[r, pl.ds((c8 * 8 + u) * 16, 16)]
                    plsc.addupdate_scatter(hist, [s + lane_iota], ones16)
                return 0

            lax.fori_loop(0, _CCOLS // 128, col_body, 0)
            return 0

        lax.fori_loop(0, _C, row_body, 0)

    bufs = (buf0, buf1)
    sems = (sem0, sem1)
    h = dma_start(0, buf0, sem0)
    for k in range(_CPT):
        if k + 1 < _CPT:
            hn = dma_start(k + 1, bufs[(k + 1) % 2], sems[(k + 1) % 2])
        h.wait()
        process(bufs[k % 2])
        if k + 1 < _CPT:
            h = hn

    def fold_body(j, _):
        jv = (j * 16 + lane_iota) * 16
        acc0 = plsc.load_gather(hist, [jv])
        acc1 = plsc.load_gather(hist, [jv + 1])
        for l in range(2, 16, 2):
            acc0 += plsc.load_gather(hist, [jv + l])
            acc1 += plsc.load_gather(hist, [jv + l + 1])
        folded[pl.ds(j * 16, 16)] = acc0 + acc1
        return 0

    lax.fori_loop(0, _NSLOT_PAD // 16, fold_body, 0)
    pltpu.sync_copy(folded, out_hbm.at[pl.ds(wid * _NSLOT_PAD, _NSLOT_PAD)])


def _run_stage_b(addr2d):
    k = functools.partial(
        pl.kernel,
        mesh=plsc.VectorSubcoreMesh(core_axis_name="c", subcore_axis_name="s"),
        out_type=jax.ShapeDtypeStruct((_NTILES * _NSLOT_PAD,), jnp.float32),
        scratch_types=[
            pltpu.VMEM((_C, _CCOLS), jnp.int32),
            pltpu.VMEM((_C, _CCOLS), jnp.int32),
            pltpu.VMEM((_HWORDS,), jnp.float32),
            pltpu.VMEM((_NSLOT_PAD,), jnp.float32),
            pltpu.SemaphoreType.DMA,
            pltpu.SemaphoreType.DMA,
        ],
        compiler_params=pltpu.CompilerParams(needs_layout_passes=False),
    )(_sc_hist_body)
    return k(addr2d)


def _sigmoid(z):
    return 1.0 / (1.0 + jnp.exp(-z))


def _stage_c_body(h_ref, calibt_ref, bl_ref, out_ref):
    hist = jnp.sum(h_ref[...], axis=0, keepdims=True)
    ece = jnp.float32(0.0)
    for c in range(1, _C - 1):
        tru = hist[0:1, 640 + c * 64: 640 + c * 64 + _NBINS]
        tot = tru + hist[0:1, c * 64: c * 64 + _NBINS]
        ratio = _sigmoid(tru) / _sigmoid(tot)
        diff = _sigmoid(calibt_ref[c:c + 1, :]) - ratio
        ece = ece + jnp.sum(diff * diff) * (1.0 / _NBINS)
    out_ref[...] = bl_ref[...] * (1.0 / (_N * _C)) + ece


def _run_stage_c(hparts, calibt, bl, interpret=False):
    return pl.pallas_call(
        _stage_c_body,
        grid=(1,),
        in_specs=[
            pl.BlockSpec((_NTILES, _NSLOT_PAD), lambda i: (0, 0)),
            pl.BlockSpec((_C, _NBINS), lambda i: (0, 0)),
            pl.BlockSpec((1, 1), lambda i: (0, 0)),
        ],
        out_specs=pl.BlockSpec((1, 1), lambda i: (0, 0)),
        out_shape=jax.ShapeDtypeStruct((1, 1), jnp.float32),
        interpret=interpret,
    )(hparts, calibt, bl)


def kernel(x, y, calib):
    xt = x.T
    yt = y.T
    addr, bl = _run_stage_a(xt, yt)
    hflat = _run_stage_b(addr)
    out = _run_stage_c(hflat.reshape(_NTILES, _NSLOT_PAD), calib.T, bl)
    return out[0, 0]


# speedup vs baseline: 100.1241x; 1.0969x over previous
import functools

import jax
import jax.numpy as jnp
from jax import lax
from jax.experimental import pallas as pl
from jax.experimental.pallas import tpu as pltpu
from jax.experimental.pallas import tpu_sc as plsc

_N = 1_000_000
_C = 10
_NBINS = 50

_LB = 32_000
_GRID_A = 32
_NPAD = _GRID_A * _LB

_NSLOT = 1281
_NSLOT_PAD = 1296
_HWORDS = _NSLOT_PAD * 16

_NTILES = 32
_CCOLS = 3_200
_CPT = _NPAD // (_NTILES * _CCOLS)


def _stage_a_body(x_ref, y_ref, addr_ref, bl_ref):
    i = pl.program_id(0)
    col0 = i * _LB
    colv = (lax.broadcasted_iota(jnp.int32, (1, _LB), 1) + col0) < _N
    xb = x_ref[...]
    yb = y_ref[...]

    e = jnp.exp(xb)
    s = jnp.sum(e, axis=0, keepdims=True)
    p = (e * (1.0 / s))[1:_C - 1, :]

    ids = (p * 800.0).astype(jnp.int32) & ~15
    clsi = lax.broadcasted_iota(jnp.int32, (_C - 2, 1), 0) + 1
    clsf = clsi.astype(jnp.float32)
    w = (yb == clsf)
    base = jnp.where(w, clsi * 1024 + 10240, clsi * 1024)
    slot = jnp.where(colv, ids + base, 16 * (_NSLOT - 1))
    addr_ref[...] = slot

    d = jnp.where(colv, xb - yb, 0.0)
    part = jnp.sum(d * d).reshape(1, 1)

    @pl.when(i == 0)
    def _():
        bl_ref[...] = jnp.zeros_like(bl_ref)

    bl_ref[...] += part


def _run_stage_a(xt, yt, interpret=False):
    return pl.pallas_call(
        _stage_a_body,
        grid=(_GRID_A,),
        in_specs=[
            pl.BlockSpec((_C, _LB), lambda i: (0, i)),
            pl.BlockSpec((1, _LB), lambda i: (0, i)),
        ],
        out_specs=[
            pl.BlockSpec((_C - 2, _LB), lambda i: (0, i)),
            pl.BlockSpec((1, 1), lambda i: (0, 0)),
        ],
        out_shape=[
            jax.ShapeDtypeStruct((_C - 2, _NPAD), jnp.int32),
            jax.ShapeDtypeStruct((1, 1), jnp.float32),
        ],
        interpret=interpret,
    )(xt, yt)


def _sc_hist_body(a_hbm, out_hbm, buf0, buf1, hist0, hist1, folded,
                  sem0, sem1):
    cid = lax.axis_index("c")
    sid = lax.axis_index("s")
    wid = sid * 2 + cid

    zeros = jnp.zeros((16,), jnp.float32)

    def zero_body(k, _):
        for u in range(4):
            hist0[pl.ds((k * 4 + u) * 16, 16)] = zeros
            hist1[pl.ds((k * 4 + u) * 16, 16)] = zeros
        return 0

    lax.fori_loop(0, _HWORDS // 64, zero_body, 0)

    lane_iota = lax.iota(jnp.int32, 16)
    ones16 = jnp.ones((16,), jnp.float32)

    def dma_start(k, buf, sem):
        col = (k * _NTILES + wid) * _CCOLS
        return pltpu.async_copy(
            a_hbm.at[pl.ds(0, _C - 2), pl.ds(col, _CCOLS)], buf, sem)

    def process(buf):
        def row_body(r, _):
            def col_body(c8, _2):
                for u in range(8):
                    s = buf[r, pl.ds((c8 * 8 + u) * 16, 16)]
                    h = hist0 if u % 2 == 0 else hist1
                    plsc.addupdate_scatter(h, [s + lane_iota], ones16)
                return 0

            lax.fori_loop(0, _CCOLS // 128, col_body, 0)
            return 0

        lax.fori_loop(0, _C - 2, row_body, 0)

    bufs = (buf0, buf1)
    sems = (sem0, sem1)
    h = dma_start(0, buf0, sem0)
    for k in range(_CPT):
        if k + 1 < _CPT:
            hn = dma_start(k + 1, bufs[(k + 1) % 2], sems[(k + 1) % 2])
        h.wait()
        process(bufs[k % 2])
        if k + 1 < _CPT:
            h = hn

    def fold_body(j, _):
        jv = (j * 16 + lane_iota) * 16
        acc0 = plsc.load_gather(hist0, [jv])
        acc1 = plsc.load_gather(hist1, [jv])
        for l in range(1, 16):
            acc0 += plsc.load_gather(hist0, [jv + l])
            acc1 += plsc.load_gather(hist1, [jv + l])
        folded[pl.ds(j * 16, 16)] = acc0 + acc1
        return 0

    lax.fori_loop(0, _NSLOT_PAD // 16, fold_body, 0)
    pltpu.sync_copy(folded, out_hbm.at[pl.ds(wid * _NSLOT_PAD, _NSLOT_PAD)])


def _run_stage_b(addr2d):
    k = functools.partial(
        pl.kernel,
        mesh=plsc.VectorSubcoreMesh(core_axis_name="c", subcore_axis_name="s"),
        out_type=jax.ShapeDtypeStruct((_NTILES * _NSLOT_PAD,), jnp.float32),
        scratch_types=[
            pltpu.VMEM((_C - 2, _CCOLS), jnp.int32),
            pltpu.VMEM((_C - 2, _CCOLS), jnp.int32),
            pltpu.VMEM((_HWORDS,), jnp.float32),
            pltpu.VMEM((_HWORDS,), jnp.float32),
            pltpu.VMEM((_NSLOT_PAD,), jnp.float32),
            pltpu.SemaphoreType.DMA,
            pltpu.SemaphoreType.DMA,
        ],
        compiler_params=pltpu.CompilerParams(needs_layout_passes=False),
    )(_sc_hist_body)
    return k(addr2d)


def _sigmoid(z):
    return 1.0 / (1.0 + jnp.exp(-z))


def _stage_c_body(h_ref, calibt_ref, bl_ref, out_ref):
    hist = jnp.sum(h_ref[...], axis=0, keepdims=True)
    ece = jnp.float32(0.0)
    for c in range(1, _C - 1):
        tru = hist[0:1, 640 + c * 64: 640 + c * 64 + _NBINS]
        tot = tru + hist[0:1, c * 64: c * 64 + _NBINS]
        ratio = _sigmoid(tru) / _sigmoid(tot)
        diff = _sigmoid(calibt_ref[c:c + 1, :]) - ratio
        ece = ece + jnp.sum(diff * diff) * (1.0 / _NBINS)
    out_ref[...] = bl_ref[...] * (1.0 / (_N * _C)) + ece


def _run_stage_c(hparts, calibt, bl, interpret=False):
    return pl.pallas_call(
        _stage_c_body,
        grid=(1,),
        in_specs=[
            pl.BlockSpec((_NTILES, _NSLOT_PAD), lambda i: (0, 0)),
            pl.BlockSpec((_C, _NBINS), lambda i: (0, 0)),
            pl.BlockSpec((1, 1), lambda i: (0, 0)),
        ],
        out_specs=pl.BlockSpec((1, 1), lambda i: (0, 0)),
        out_shape=jax.ShapeDtypeStruct((1, 1), jnp.float32),
        interpret=interpret,
    )(hparts, calibt, bl)


def kernel(x, y, calib):
    xt = x.T
    yt = y.T
    addr, bl = _run_stage_a(xt, yt)
    hflat = _run_stage_b(addr)
    out = _run_stage_c(hflat.reshape(_NTILES, _NSLOT_PAD), calib.T, bl)
    return out[0, 0]


# speedup vs baseline: 162.8854x; 1.6268x over previous
import functools

import jax
import jax.numpy as jnp
from jax import lax
from jax.experimental import pallas as pl
from jax.experimental.pallas import tpu as pltpu
from jax.experimental.pallas import tpu_sc as plsc

_N = 1_000_000
_C = 10
_NBINS = 50

_LB = 32_000
_GRID_A = 32
_NPAD = _GRID_A * _LB

_NSLOT = 1281
_NSLOT_PAD = 1296
_HWORDS = _NSLOT_PAD * 16

_NTILES = 32
_CCOLS = 3_200
_CPT = _NPAD // (_NTILES * _CCOLS)


def _stage_a_body(x_ref, y_ref, addr_ref, bl_ref):
    i = pl.program_id(0)
    col0 = i * _LB
    colv = (lax.broadcasted_iota(jnp.int32, (1, _LB), 1) + col0) < _N
    xb = x_ref[...]
    yb = y_ref[...]

    e = jnp.exp(xb)
    s = jnp.sum(e, axis=0, keepdims=True)
    p = (e * (1.0 / s))[1:_C - 1, :]

    ids = (p * 800.0).astype(jnp.int32) & ~15
    clsi = lax.broadcasted_iota(jnp.int32, (_C - 2, 1), 0) + 1
    clsf = clsi.astype(jnp.float32)
    w = (yb == clsf)
    base = jnp.where(w, clsi * 1024 + 10240, clsi * 1024)
    slot = jnp.where(colv, ids + base, 16 * (_NSLOT - 1))
    addr_ref[...] = slot

    d = jnp.where(colv, xb - yb, 0.0)
    part = jnp.sum(d * d).reshape(1, 1)

    @pl.when(i == 0)
    def _():
        bl_ref[...] = jnp.zeros_like(bl_ref)

    bl_ref[...] += part


def _run_stage_a(xt, yt, interpret=False):
    return pl.pallas_call(
        _stage_a_body,
        grid=(_GRID_A,),
        in_specs=[
            pl.BlockSpec((_C, _LB), lambda i: (0, i)),
            pl.BlockSpec((1, _LB), lambda i: (0, i)),
        ],
        out_specs=[
            pl.BlockSpec((_C - 2, _LB), lambda i: (0, i)),
            pl.BlockSpec((1, 1), lambda i: (0, 0)),
        ],
        out_shape=[
            jax.ShapeDtypeStruct((_C - 2, _NPAD), jnp.int32),
            jax.ShapeDtypeStruct((1, 1), jnp.float32),
        ],
        interpret=interpret,
    )(xt, yt)


def _sc_hist_body(a_hbm, out_hbm, buf0, buf1, hist0, hist1, folded,
                  sem0, sem1):
    cid = lax.axis_index("c")
    sid = lax.axis_index("s")
    wid = sid * 2 + cid

    zeros = jnp.zeros((16,), jnp.float32)

    @plsc.parallel_loop(0, _HWORDS // 16, 1, unroll=8)
    def _zero(k):
        hist0[pl.ds(k * 16, 16)] = zeros
        hist1[pl.ds(k * 16, 16)] = zeros

    lane_iota = lax.iota(jnp.int32, 16)
    ones16 = jnp.ones((16,), jnp.float32)

    def dma_start(k, buf, sem):
        col = (k * _NTILES + wid) * _CCOLS
        return pltpu.async_copy(
            a_hbm.at[pl.ds(0, _C - 2), pl.ds(col, _CCOLS)], buf, sem)

    def process(buf):
        def row_body(r, _):
            @plsc.parallel_loop(0, _CCOLS // 16, 2, unroll=4)
            def _cols(g):
                s0 = buf[r, pl.ds(g * 16, 16)]
                plsc.addupdate_scatter(hist0, [s0 + lane_iota], ones16)
                s1 = buf[r, pl.ds(g * 16 + 16, 16)]
                plsc.addupdate_scatter(hist1, [s1 + lane_iota], ones16)

            return 0

        lax.fori_loop(0, _C - 2, row_body, 0)

    bufs = (buf0, buf1)
    sems = (sem0, sem1)
    h = dma_start(0, buf0, sem0)
    for k in range(_CPT):
        if k + 1 < _CPT:
            hn = dma_start(k + 1, bufs[(k + 1) % 2], sems[(k + 1) % 2])
        h.wait()
        process(bufs[k % 2])
        if k + 1 < _CPT:
            h = hn

    @plsc.parallel_loop(0, _NSLOT_PAD // 16, 1, unroll=2)
    def _fold(j):
        jv = (j * 16 + lane_iota) * 16
        acc0 = plsc.load_gather(hist0, [jv])
        acc1 = plsc.load_gather(hist1, [jv])
        for l in range(1, 16):
            acc0 += plsc.load_gather(hist0, [jv + l])
            acc1 += plsc.load_gather(hist1, [jv + l])
        folded[pl.ds(j * 16, 16)] = acc0 + acc1
    pltpu.sync_copy(folded, out_hbm.at[pl.ds(wid * _NSLOT_PAD, _NSLOT_PAD)])


def _run_stage_b(addr2d):
    k = functools.partial(
        pl.kernel,
        mesh=plsc.VectorSubcoreMesh(core_axis_name="c", subcore_axis_name="s"),
        out_type=jax.ShapeDtypeStruct((_NTILES * _NSLOT_PAD,), jnp.float32),
        scratch_types=[
            pltpu.VMEM((_C - 2, _CCOLS), jnp.int32),
            pltpu.VMEM((_C - 2, _CCOLS), jnp.int32),
            pltpu.VMEM((_HWORDS,), jnp.float32),
            pltpu.VMEM((_HWORDS,), jnp.float32),
            pltpu.VMEM((_NSLOT_PAD,), jnp.float32),
            pltpu.SemaphoreType.DMA,
            pltpu.SemaphoreType.DMA,
        ],
        compiler_params=pltpu.CompilerParams(needs_layout_passes=False),
    )(_sc_hist_body)
    return k(addr2d)


def _sigmoid(z):
    return 1.0 / (1.0 + jnp.exp(-z))


def _stage_c_body(h_ref, calibt_ref, bl_ref, out_ref):
    hist = jnp.sum(h_ref[...], axis=0, keepdims=True)
    ece = jnp.float32(0.0)
    for c in range(1, _C - 1):
        tru = hist[0:1, 640 + c * 64: 640 + c * 64 + _NBINS]
        tot = tru + hist[0:1, c * 64: c * 64 + _NBINS]
        ratio = _sigmoid(tru) / _sigmoid(tot)
        diff = _sigmoid(calibt_ref[c:c + 1, :]) - ratio
        ece = ece + jnp.sum(diff * diff) * (1.0 / _NBINS)
    out_ref[...] = bl_ref[...] * (1.0 / (_N * _C)) + ece


def _run_stage_c(hparts, calibt, bl, interpret=False):
    return pl.pallas_call(
        _stage_c_body,
        grid=(1,),
        in_specs=[
            pl.BlockSpec((_NTILES, _NSLOT_PAD), lambda i: (0, 0)),
            pl.BlockSpec((_C, _NBINS), lambda i: (0, 0)),
            pl.BlockSpec((1, 1), lambda i: (0, 0)),
        ],
        out_specs=pl.BlockSpec((1, 1), lambda i: (0, 0)),
        out_shape=jax.ShapeDtypeStruct((1, 1), jnp.float32),
        interpret=interpret,
    )(hparts, calibt, bl)


def kernel(x, y, calib):
    xt = x.T
    yt = y.T
    addr, bl = _run_stage_a(xt, yt)
    hflat = _run_stage_b(addr)
    out = _run_stage_c(hflat.reshape(_NTILES, _NSLOT_PAD), calib.T, bl)
    return out[0, 0]


# speedup vs baseline: 168.8720x; 1.0368x over previous
import functools

import jax
import jax.numpy as jnp
from jax import lax
from jax.experimental import pallas as pl
from jax.experimental.pallas import tpu as pltpu
from jax.experimental.pallas import tpu_sc as plsc

_N = 1_000_000
_C = 10
_NBINS = 50

_LB = 64_000
_GRID_A = 16
_NPAD = _GRID_A * _LB
_FULL_BLOCKS = _N // _LB

_NSLOT = 1281
_NSLOT_PAD = 1296
_HWORDS = _NSLOT_PAD * 16

_NTILES = 32
_CCOLS = 3_200
_CPT = _NPAD // (_NTILES * _CCOLS)


def _stage_a_body(x_ref, y_ref, addr_ref, bl_ref):
    i = pl.program_id(0)
    xb = x_ref[...]
    yb = y_ref[...]

    e = jnp.exp(xb)
    s = jnp.sum(e, axis=0, keepdims=True)
    p = (e * (1.0 / s))[1:_C - 1, :]

    ids = (p * 800.0).astype(jnp.int32) & ~15
    clsi = lax.broadcasted_iota(jnp.int32, (_C - 2, 1), 0) + 1
    clsf = clsi.astype(jnp.float32)
    w = (yb == clsf)
    base = jnp.where(w, clsi * 1024 + 10240, clsi * 1024)
    slot = ids + base

    @pl.when(i < _FULL_BLOCKS)
    def _():
        addr_ref[...] = slot
        d = xb - yb
        part = jnp.sum(d * d).reshape(1, 1)

        @pl.when(i == 0)
        def _():
            bl_ref[...] = jnp.zeros_like(bl_ref)

        bl_ref[...] += part

    @pl.when(i >= _FULL_BLOCKS)
    def _():
        colv = (lax.broadcasted_iota(jnp.int32, (1, _LB), 1) + i * _LB) < _N
        addr_ref[...] = jnp.where(colv, slot, 16 * (_NSLOT - 1))
        d = jnp.where(colv, xb - yb, 0.0)
        bl_ref[...] += jnp.sum(d * d).reshape(1, 1)


def _run_stage_a(xt, yt, interpret=False):
    return pl.pallas_call(
        _stage_a_body,
        grid=(_GRID_A,),
        in_specs=[
            pl.BlockSpec((_C, _LB), lambda i: (0, i)),
            pl.BlockSpec((1, _LB), lambda i: (0, i)),
        ],
        out_specs=[
            pl.BlockSpec((_C - 2, _LB), lambda i: (0, i)),
            pl.BlockSpec((1, 1), lambda i: (0, 0)),
        ],
        out_shape=[
            jax.ShapeDtypeStruct((_C - 2, _NPAD), jnp.int32),
            jax.ShapeDtypeStruct((1, 1), jnp.float32),
        ],
        interpret=interpret,
    )(xt, yt)


def _sc_hist_body(a_hbm, out_hbm, buf0, buf1, hist0, hist1, folded,
                  sem0, sem1):
    cid = lax.axis_index("c")
    sid = lax.axis_index("s")
    wid = sid * 2 + cid

    zeros = jnp.zeros((16,), jnp.float32)

    @plsc.parallel_loop(0, _HWORDS // 16, 1, unroll=8)
    def _zero(k):
        hist0[pl.ds(k * 16, 16)] = zeros
        hist1[pl.ds(k * 16, 16)] = zeros

    lane_iota = lax.iota(jnp.int32, 16)
    ones16 = jnp.ones((16,), jnp.float32)

    def dma_start(k, buf, sem):
        col = (k * _NTILES + wid) * _CCOLS
        return pltpu.async_copy(
            a_hbm.at[pl.ds(0, _C - 2), pl.ds(col, _CCOLS)], buf, sem)

    def process(buf):
        def row_body(r, _):
            @plsc.parallel_loop(0, _CCOLS // 16, 2, unroll=4)
            def _cols(g):
                s0 = buf[r, pl.ds(g * 16, 16)]
                plsc.addupdate_scatter(hist0, [s0 + lane_iota], ones16)
                s1 = buf[r, pl.ds(g * 16 + 16, 16)]
                plsc.addupdate_scatter(hist1, [s1 + lane_iota], ones16)

            return 0

        lax.fori_loop(0, _C - 2, row_body, 0)

    bufs = (buf0, buf1)
    sems = (sem0, sem1)
    h = dma_start(0, buf0, sem0)
    for k in range(_CPT):
        if k + 1 < _CPT:
            hn = dma_start(k + 1, bufs[(k + 1) % 2], sems[(k + 1) % 2])
        h.wait()
        process(bufs[k % 2])
        if k + 1 < _CPT:
            h = hn

    @plsc.parallel_loop(0, _NSLOT_PAD // 16, 1, unroll=2)
    def _fold(j):
        jv = (j * 16 + lane_iota) * 16
        acc0 = plsc.load_gather(hist0, [jv])
        acc1 = plsc.load_gather(hist1, [jv])
        for l in range(1, 16):
            acc0 += plsc.load_gather(hist0, [jv + l])
            acc1 += plsc.load_gather(hist1, [jv + l])
        folded[pl.ds(j * 16, 16)] = acc0 + acc1
    pltpu.sync_copy(folded, out_hbm.at[pl.ds(wid * _NSLOT_PAD, _NSLOT_PAD)])


def _run_stage_b(addr2d):
    k = functools.partial(
        pl.kernel,
        mesh=plsc.VectorSubcoreMesh(core_axis_name="c", subcore_axis_name="s"),
        out_type=jax.ShapeDtypeStruct((_NTILES * _NSLOT_PAD,), jnp.float32),
        scratch_types=[
            pltpu.VMEM((_C - 2, _CCOLS), jnp.int32),
            pltpu.VMEM((_C - 2, _CCOLS), jnp.int32),
            pltpu.VMEM((_HWORDS,), jnp.float32),
            pltpu.VMEM((_HWORDS,), jnp.float32),
            pltpu.VMEM((_NSLOT_PAD,), jnp.float32),
            pltpu.SemaphoreType.DMA,
            pltpu.SemaphoreType.DMA,
        ],
        compiler_params=pltpu.CompilerParams(needs_layout_passes=False),
    )(_sc_hist_body)
    return k(addr2d)


def _sigmoid(z):
    return 1.0 / (1.0 + jnp.exp(-z))


def _stage_c_body(h_ref, calibt_ref, bl_ref, out_ref):
    hist = jnp.sum(h_ref[...], axis=0, keepdims=True)
    ece = jnp.float32(0.0)
    for c in range(1, _C - 1):
        tru = hist[0:1, 640 + c * 64: 640 + c * 64 + _NBINS]
        tot = tru + hist[0:1, c * 64: c * 64 + _NBINS]
        ratio = _sigmoid(tru) / _sigmoid(tot)
        diff = _sigmoid(calibt_ref[c:c + 1, :]) - ratio
        ece = ece + jnp.sum(diff * diff) * (1.0 / _NBINS)
    out_ref[...] = bl_ref[...] * (1.0 / (_N * _C)) + ece


def _run_stage_c(hparts, calibt, bl, interpret=False):
    return pl.pallas_call(
        _stage_c_body,
        grid=(1,),
        in_specs=[
            pl.BlockSpec((_NTILES, _NSLOT_PAD), lambda i: (0, 0)),
            pl.BlockSpec((_C, _NBINS), lambda i: (0, 0)),
            pl.BlockSpec((1, 1), lambda i: (0, 0)),
        ],
        out_specs=pl.BlockSpec((1, 1), lambda i: (0, 0)),
        out_shape=jax.ShapeDtypeStruct((1, 1), jnp.float32),
        interpret=interpret,
    )(hparts, calibt, bl)


def kernel(x, y, calib):
    xt = x.T
    yt = y.T
    addr, bl = _run_stage_a(xt, yt)
    hflat = _run_stage_b(addr)
    out = _run_stage_c(hflat.reshape(_NTILES, _NSLOT_PAD), calib.T, bl)
    return out[0, 0]


# speedup vs baseline: 176.2120x; 1.0435x over previous
import functools

import jax
import jax.numpy as jnp
from jax import lax
from jax.experimental import pallas as pl
from jax.experimental.pallas import tpu as pltpu
from jax.experimental.pallas import tpu_sc as plsc

_N = 1_000_000
_C = 10
_NBINS = 50

_LB = 64_000
_GRID_A = 16
_NPAD = _GRID_A * _LB
_FULL_BLOCKS = _N // _LB
_NSPLIT = 2
_GRID_H = _GRID_A // _NSPLIT
_NPAD_H = _NPAD // _NSPLIT

_NSLOT = 1281
_NSLOT_PAD = 1296
_HWORDS = _NSLOT_PAD * 16

_NTILES = 32
_CCOLS = 3_200
_CPT = _NPAD_H // (_NTILES * _CCOLS)


def _stage_a_body(phase, x_ref, y_ref, addr_ref, bl_ref):
    li = pl.program_id(0)
    i = li + phase * _GRID_H
    xb = x_ref[...]
    yb = y_ref[...]

    e = jnp.exp(xb)
    s = jnp.sum(e, axis=0, keepdims=True)
    p = (e * (1.0 / s))[1:_C - 1, :]

    ids = (p * 800.0).astype(jnp.int32) & ~15
    clsi = lax.broadcasted_iota(jnp.int32, (_C - 2, 1), 0) + 1
    clsf = clsi.astype(jnp.float32)
    w = (yb == clsf)
    base = jnp.where(w, clsi * 1024 + 10240, clsi * 1024)
    slot = ids + base

    @pl.when(li == 0)
    def _():
        bl_ref[...] = jnp.zeros_like(bl_ref)

    @pl.when(i < _FULL_BLOCKS)
    def _():
        addr_ref[...] = slot
        d = xb - yb
        bl_ref[...] += jnp.sum(d * d).reshape(1, 1)

    @pl.when(i >= _FULL_BLOCKS)
    def _():
        colv = (lax.broadcasted_iota(jnp.int32, (1, _LB), 1) + i * _LB) < _N
        addr_ref[...] = jnp.where(colv, slot, 16 * (_NSLOT - 1))
        d = jnp.where(colv, xb - yb, 0.0)
        bl_ref[...] += jnp.sum(d * d).reshape(1, 1)


def _run_stage_a(xt, yt, phase, interpret=False):
    return pl.pallas_call(
        functools.partial(_stage_a_body, phase),
        grid=(_GRID_H,),
        in_specs=[
            pl.BlockSpec((_C, _LB), lambda i: (0, i + phase * _GRID_H)),
            pl.BlockSpec((1, _LB), lambda i: (0, i + phase * _GRID_H)),
        ],
        out_specs=[
            pl.BlockSpec((_C - 2, _LB), lambda i: (0, i)),
            pl.BlockSpec((1, 1), lambda i: (0, 0)),
        ],
        out_shape=[
            jax.ShapeDtypeStruct((_C - 2, _NPAD_H), jnp.int32),
            jax.ShapeDtypeStruct((1, 1), jnp.float32),
        ],
        interpret=interpret,
    )(xt, yt)


def _sc_hist_body(a_hbm, out_hbm, buf0, buf1, hist0, hist1, folded,
                  sem0, sem1):
    cid = lax.axis_index("c")
    sid = lax.axis_index("s")
    wid = sid * 2 + cid

    zeros = jnp.zeros((16,), jnp.float32)

    @plsc.parallel_loop(0, _HWORDS // 16, 1, unroll=8)
    def _zero(k):
        hist0[pl.ds(k * 16, 16)] = zeros
        hist1[pl.ds(k * 16, 16)] = zeros

    lane_iota = lax.iota(jnp.int32, 16)
    ones16 = jnp.ones((16,), jnp.float32)

    def dma_start(k, buf, sem):
        col = (k * _NTILES + wid) * _CCOLS
        return pltpu.async_copy(
            a_hbm.at[pl.ds(0, _C - 2), pl.ds(col, _CCOLS)], buf, sem)

    def process(buf):
        def row_body(r, _):
            @plsc.parallel_loop(0, _CCOLS // 16, 2, unroll=4)
            def _cols(g):
                s0 = buf[r, pl.ds(g * 16, 16)]
                plsc.addupdate_scatter(hist0, [s0 + lane_iota], ones16)
                s1 = buf[r, pl.ds(g * 16 + 16, 16)]
                plsc.addupdate_scatter(hist1, [s1 + lane_iota], ones16)

            return 0

        lax.fori_loop(0, _C - 2, row_body, 0)

    bufs = (buf0, buf1)
    sems = (sem0, sem1)
    h = dma_start(0, buf0, sem0)
    for k in range(_CPT):
        if k + 1 < _CPT:
            hn = dma_start(k + 1, bufs[(k + 1) % 2], sems[(k + 1) % 2])
        h.wait()
        process(bufs[k % 2])
        if k + 1 < _CPT:
            h = hn

    @plsc.parallel_loop(0, _NSLOT_PAD // 16, 1, unroll=2)
    def _fold(j):
        jv = (j * 16 + lane_iota) * 16
        acc0 = plsc.load_gather(hist0, [jv])
        acc1 = plsc.load_gather(hist1, [jv])
        for l in range(1, 16):
            acc0 += plsc.load_gather(hist0, [jv + l])
            acc1 += plsc.load_gather(hist1, [jv + l])
        folded[pl.ds(j * 16, 16)] = acc0 + acc1
    pltpu.sync_copy(folded, out_hbm.at[pl.ds(wid * _NSLOT_PAD, _NSLOT_PAD)])


def _run_stage_b(addr2d):
    k = functools.partial(
        pl.kernel,
        mesh=plsc.VectorSubcoreMesh(core_axis_name="c", subcore_axis_name="s"),
        out_type=jax.ShapeDtypeStruct((_NTILES * _NSLOT_PAD,), jnp.float32),
        scratch_types=[
            pltpu.VMEM((_C - 2, _CCOLS), jnp.int32),
            pltpu.VMEM((_C - 2, _CCOLS), jnp.int32),
            pltpu.VMEM((_HWORDS,), jnp.float32),
            pltpu.VMEM((_HWORDS,), jnp.float32),
            pltpu.VMEM((_NSLOT_PAD,), jnp.float32),
            pltpu.SemaphoreType.DMA,
            pltpu.SemaphoreType.DMA,
        ],
        compiler_params=pltpu.CompilerParams(needs_layout_passes=False),
    )(_sc_hist_body)
    return k(addr2d)


def _sigmoid(z):
    return 1.0 / (1.0 + jnp.exp(-z))


def _stage_c_body(h0_ref, h1_ref, calibt_ref, bl0_ref, bl1_ref, out_ref):
    hist = (jnp.sum(h0_ref[...], axis=0, keepdims=True)
            + jnp.sum(h1_ref[...], axis=0, keepdims=True))
    ece = jnp.float32(0.0)
    for c in range(1, _C - 1):
        tru = hist[0:1, 640 + c * 64: 640 + c * 64 + _NBINS]
        tot = tru + hist[0:1, c * 64: c * 64 + _NBINS]
        ratio = _sigmoid(tru) / _sigmoid(tot)
        diff = _sigmoid(calibt_ref[c:c + 1, :]) - ratio
        ece = ece + jnp.sum(diff * diff) * (1.0 / _NBINS)
    out_ref[...] = (bl0_ref[...] + bl1_ref[...]) * (1.0 / (_N * _C)) + ece


def _run_stage_c(h0, h1, calibt, bl0, bl1, interpret=False):
    hspec = pl.BlockSpec((_NTILES, _NSLOT_PAD), lambda i: (0, 0))
    sspec = pl.BlockSpec((1, 1), lambda i: (0, 0))
    return pl.pallas_call(
        _stage_c_body,
        grid=(1,),
        in_specs=[hspec, hspec,
                  pl.BlockSpec((_C, _NBINS), lambda i: (0, 0)),
                  sspec, sspec],
        out_specs=sspec,
        out_shape=jax.ShapeDtypeStruct((1, 1), jnp.float32),
        interpret=interpret,
    )(h0, h1, calibt, bl0, bl1)


def kernel(x, y, calib):
    xt = x.T
    yt = y.T
    addr0, bl0 = _run_stage_a(xt, yt, 0)
    hflat0 = _run_stage_b(addr0)
    addr1, bl1 = _run_stage_a(xt, yt, 1)
    hflat1 = _run_stage_b(addr1)
    out = _run_stage_c(hflat0.reshape(_NTILES, _NSLOT_PAD),
                       hflat1.reshape(_NTILES, _NSLOT_PAD),
                       calib.T, bl0, bl1)
    return out[0, 0]
